# baseline (speedup 1.0000x reference)
import functools

import jax
import jax.numpy as jnp
from jax.experimental import pallas as pl
from jax.experimental.pallas import tpu as pltpu

_BK = 8192
_BQ = 1024


def _thr_kernel(sample_ref, positive_ref, thr_ref):
    s = sample_ref[...]
    t = s - positive_ref[...] + 1e-6
    t2 = jnp.sum(t * t, axis=1, keepdims=True)
    a2 = jnp.sum(s * s, axis=1, keepdims=True)
    thr_ref[...] = jnp.where(t2 >= 1e-12, 0.5 * (t2 - a2), -1e30)


def _count_kernel(sample_ref, thr_ref, gal_ref, acc_ref, *, bk, nk, rem):
    j = pl.program_id(1)

    @pl.when(j == 0)
    def _init():
        acc_ref[...] = jnp.zeros_like(acc_ref)

    g = gal_ref[...]
    negh = jnp.full((8, g.shape[1]), -0.5, jnp.bfloat16)
    nb = jax.lax.dot_general(
        negh, g * g,
        dimension_numbers=(((1,), (1,)), ((), ())),
        preferred_element_type=jnp.float32)
    ab = jax.lax.dot_general(
        sample_ref[...], g,
        dimension_numbers=(((1,), (1,)), ((), ())),
        preferred_element_type=jnp.float32)
    z = (ab + nb[0:1, :]) + thr_ref[...]

    def _accum(zv):
        zb = zv.astype(jnp.bfloat16)
        neg = jnp.where(zb < 0, jnp.bfloat16(1), jnp.bfloat16(0))
        partial = neg[:, 0:128]
        for c in range(1, bk // 128):
            partial = partial + neg[:, c * 128:(c + 1) * 128]
        acc_ref[...] = acc_ref[...] + partial.astype(jnp.float32)

    @pl.when(j < nk - 1)
    def _full():
        _accum(z)

    @pl.when(j == nk - 1)
    def _tail():
        col = jax.lax.broadcasted_iota(jnp.int32, z.shape, 1)
        _accum(jnp.where(col < rem, z, -1.0))


def _fin_kernel(acc_ref, out_ref, *, k_pad):
    rank = k_pad - jnp.sum(acc_ref[...], axis=1).astype(jnp.int32)
    out_ref[...] = jnp.maximum(rank, 1)


def _ranks(sample_feature, positive_feature, gallery):
    q, d = sample_feature.shape
    k = gallery.shape[0]
    bk = min(_BK, k)
    nk = pl.cdiv(k, bk)
    k_pad = nk * bk
    rem = k - (nk - 1) * bk

    thr = pl.pallas_call(
        _thr_kernel,
        out_shape=jax.ShapeDtypeStruct((q, 1), jnp.float32),
    )(sample_feature, positive_feature)

    bq = _BQ if q % _BQ == 0 else q
    nq = q // bq
    acc = pl.pallas_call(
        functools.partial(_count_kernel, bk=bk, nk=nk, rem=rem),
        grid=(nq, nk),
        in_specs=[
            pl.BlockSpec((bq, d), lambda i, j: (i, 0)),
            pl.BlockSpec((bq, 1), lambda i, j: (i, 0)),
            pl.BlockSpec((bk, d), lambda i, j: (j, 0)),
        ],
        out_specs=pl.BlockSpec((bq, 128), lambda i, j: (i, 0)),
        out_shape=jax.ShapeDtypeStruct((q, 128), jnp.float32),
        compiler_params=pltpu.CompilerParams(
            dimension_semantics=("parallel", "arbitrary")),
    )(sample_feature, thr, gallery)

    return pl.pallas_call(
        functools.partial(_fin_kernel, k_pad=k_pad),
        out_shape=jax.ShapeDtypeStruct((q,), jnp.int32),
    )(acc)


def kernel(sample_feature, positive_feature, gallery):
    rank = _ranks(sample_feature, positive_feature, gallery)
    rank_f = rank.astype(jnp.float32)
    top1 = jnp.mean((rank <= 1).astype(jnp.float32))
    top10 = jnp.mean((rank <= 10).astype(jnp.float32))
    avg = jnp.mean(rank_f)
    return (rank, top1, top10, avg)


# speedup vs baseline: 1.0057x; 1.0057x over previous
import functools

import jax
import jax.numpy as jnp
from jax.experimental import pallas as pl
from jax.experimental.pallas import tpu as pltpu

_BK = 4096
_BQ = 1024


def _thr_kernel(sample_ref, positive_ref, thr_ref):
    s = sample_ref[...]
    t = s - positive_ref[...] + 1e-6
    t2 = jnp.sum(t * t, axis=1, keepdims=True)
    a2 = jnp.sum(s * s, axis=1, keepdims=True)
    thr_ref[...] = jnp.where(t2 >= 1e-12, 0.5 * (t2 - a2), -1e30)


def _count_kernel(sample_ref, thr_ref, gal_ref, acc_ref, *, bk, nk, rem):
    j = pl.program_id(1)

    @pl.when(j == 0)
    def _init():
        acc_ref[...] = jnp.zeros_like(acc_ref)

    g = gal_ref[...]
    negh = jnp.full((8, g.shape[1]), -0.5, jnp.bfloat16)
    nb = jax.lax.dot_general(
        negh, g * g,
        dimension_numbers=(((1,), (1,)), ((), ())),
        preferred_element_type=jnp.float32)
    ab = jax.lax.dot_general(
        sample_ref[...], g,
        dimension_numbers=(((1,), (1,)), ((), ())),
        preferred_element_type=jnp.float32)
    z = (ab + nb[0:1, :]) + thr_ref[...]

    def _accum(zv):
        zb = zv.astype(jnp.bfloat16)
        neg = jnp.where(zb < 0, jnp.bfloat16(1), jnp.bfloat16(0))
        partial = neg[:, 0:128]
        for c in range(1, bk // 128):
            partial = partial + neg[:, c * 128:(c + 1) * 128]
        acc_ref[...] = acc_ref[...] + partial.astype(jnp.float32)

    @pl.when(j < nk - 1)
    def _full():
        _accum(z)

    @pl.when(j == nk - 1)
    def _tail():
        col = jax.lax.broadcasted_iota(jnp.int32, z.shape, 1)
        _accum(jnp.where(col < rem, z, -1.0))


def _fin_kernel(acc_ref, out_ref, *, k_pad):
    rank = k_pad - jnp.sum(acc_ref[...], axis=1).astype(jnp.int32)
    out_ref[...] = jnp.maximum(rank, 1)


def _ranks(sample_feature, positive_feature, gallery):
    q, d = sample_feature.shape
    k = gallery.shape[0]
    bk = min(_BK, k)
    nk = pl.cdiv(k, bk)
    k_pad = nk * bk
    rem = k - (nk - 1) * bk

    thr = pl.pallas_call(
        _thr_kernel,
        out_shape=jax.ShapeDtypeStruct((q, 1), jnp.float32),
    )(sample_feature, positive_feature)

    bq = _BQ if q % _BQ == 0 else q
    nq = q // bq
    acc = pl.pallas_call(
        functools.partial(_count_kernel, bk=bk, nk=nk, rem=rem),
        grid=(nq, nk),
        in_specs=[
            pl.BlockSpec((bq, d), lambda i, j: (i, 0)),
            pl.BlockSpec((bq, 1), lambda i, j: (i, 0)),
            pl.BlockSpec((bk, d), lambda i, j: (j, 0)),
        ],
        out_specs=pl.BlockSpec((bq, 128), lambda i, j: (i, 0)),
        out_shape=jax.ShapeDtypeStruct((q, 128), jnp.float32),
        compiler_params=pltpu.CompilerParams(
            dimension_semantics=("parallel", "arbitrary")),
    )(sample_feature, thr, gallery)

    return pl.pallas_call(
        functools.partial(_fin_kernel, k_pad=k_pad),
        out_shape=jax.ShapeDtypeStruct((q,), jnp.int32),
    )(acc)


def kernel(sample_feature, positive_feature, gallery):
    rank = _ranks(sample_feature, positive_feature, gallery)
    rank_f = rank.astype(jnp.float32)
    top1 = jnp.mean((rank <= 1).astype(jnp.float32))
    top10 = jnp.mean((rank <= 10).astype(jnp.float32))
    avg = jnp.mean(rank_f)
    return (rank, top1, top10, avg)
